# baseline (speedup 1.0000x reference)
import functools

import jax
import jax.numpy as jnp
from jax import lax
from jax.experimental import pallas as pl
from jax.experimental.pallas import tpu as pltpu
from jax.experimental.pallas import tpu_sc as plsc

D_MODEL = 64
SCALE = 8.0

NC, NS, LANES = 2, 16, 16
NW = NC * NS
B_TOTAL = 4096 * 200
BPW = B_TOTAL // NW
IDX_MINOR = 128
CHUNK = 512
K = CHUNK // IDX_MINOR
G = BPW // CHUNK
IDX_ROWS = BPW // IDX_MINOR

_mesh = plsc.VectorSubcoreMesh(core_axis_name="c", subcore_axis_name="s")


@functools.partial(
    pl.kernel,
    out_type=jax.ShapeDtypeStruct((B_TOTAL, D_MODEL), jnp.float32),
    mesh=_mesh,
    scratch_types=[
        pltpu.VMEM((IDX_ROWS, IDX_MINOR), jnp.int32),
        pltpu.VMEM((CHUNK, D_MODEL), jnp.float32),
        pltpu.SemaphoreType.DMA,
    ],
    compiler_params=pltpu.CompilerParams(use_tc_tiling_on_sc=False),
)
def _embed_sc(x_hbm, tab_hbm, out_hbm, idx_v, buf, sem):
    wid = lax.axis_index("s") * NC + lax.axis_index("c")
    pltpu.sync_copy(x_hbm.at[pl.ds(wid * IDX_ROWS, IDX_ROWS)], idx_v)

    def chunk_body(g, carry):
        copies = [
            pltpu.async_copy(
                tab_hbm.at[idx_v.at[g * K + j]],
                buf.at[pl.ds(j * IDX_MINOR, IDX_MINOR)],
                sem,
            )
            for j in range(K)
        ]
        for cp in copies:
            cp.wait()

        def scale_row(r, c):
            for j in range(D_MODEL // LANES):
                sl = pl.ds(j * LANES, LANES)
                buf[r, sl] = buf[r, sl] * SCALE
            return c

        lax.fori_loop(0, CHUNK, scale_row, 0)
        pltpu.sync_copy(buf, out_hbm.at[pl.ds(wid * BPW + g * CHUNK, CHUNK)])
        return carry

    lax.fori_loop(0, G, chunk_body, 0)


def kernel(x, table):
    xi = x.astype(jnp.int32).reshape(NW * IDX_ROWS, IDX_MINOR)
    out = _embed_sc(xi, table)
    return out.reshape(x.shape[0], x.shape[1], D_MODEL)


# speedup vs baseline: 1.1132x; 1.1132x over previous
import functools

import jax
import jax.numpy as jnp
from jax import lax
from jax.experimental import pallas as pl
from jax.experimental.pallas import tpu as pltpu
from jax.experimental.pallas import tpu_sc as plsc

D_MODEL = 64
SCALE = 8.0

NC, NS, LANES = 2, 16, 16
NW = NC * NS
B_TOTAL = 4096 * 200
BPW = B_TOTAL // NW
IDX_MINOR = 128
CHUNK = 256
K = CHUNK // IDX_MINOR
G = BPW // CHUNK
IDX_ROWS = BPW // IDX_MINOR
NBUF = 4
LOOKAHEAD = 2

_mesh = plsc.VectorSubcoreMesh(core_axis_name="c", subcore_axis_name="s")


@functools.partial(
    pl.kernel,
    out_type=jax.ShapeDtypeStruct((B_TOTAL, D_MODEL), jnp.float32),
    mesh=_mesh,
    scratch_types=[
        pltpu.VMEM((IDX_ROWS, IDX_MINOR), jnp.int32),
        [pltpu.VMEM((CHUNK, D_MODEL), jnp.float32)] * NBUF,
        [pltpu.SemaphoreType.DMA] * NBUF,
        [pltpu.SemaphoreType.DMA] * NBUF,
    ],
    compiler_params=pltpu.CompilerParams(use_tc_tiling_on_sc=False),
)
def _embed_sc(x_hbm, tab_hbm, out_hbm, idx_v, bufs, gsems, osems):
    wid = lax.axis_index("s") * NC + lax.axis_index("c")
    row0 = wid * BPW
    pltpu.sync_copy(x_hbm.at[pl.ds(wid * IDX_ROWS, IDX_ROWS)], idx_v)

    def issue_gather(f, b):
        for j in range(K):
            pltpu.async_copy(
                tab_hbm.at[idx_v.at[f * K + j]],
                bufs[b].at[pl.ds(j * IDX_MINOR, IDX_MINOR)],
                gsems[b],
            )

    def wait_gather(b):
        for j in range(K):
            pltpu.make_async_copy(
                tab_hbm.at[idx_v.at[0]],
                bufs[b].at[pl.ds(j * IDX_MINOR, IDX_MINOR)],
                gsems[b],
            ).wait()

    def wait_out(b):
        pltpu.make_async_copy(
            bufs[b], out_hbm.at[pl.ds(0, CHUNK)], osems[b]
        ).wait()

    def scale_buf(b):
        @plsc.parallel_loop(0, CHUNK, step=4)
        def _scale(r):
            for rr in range(4):
                for j in range(D_MODEL // LANES):
                    sl = pl.ds(j * LANES, LANES)
                    bufs[b][r + rr, sl] = bufs[b][r + rr, sl] * SCALE

    def consume(g, b):
        wait_gather(b)
        scale_buf(b)
        pltpu.async_copy(
            bufs[b], out_hbm.at[pl.ds(row0 + g * CHUNK, CHUNK)], osems[b]
        )

    def visit(g, b, drain_out):
        bf = (b + LOOKAHEAD) % NBUF
        if drain_out:
            wait_out(bf)
        issue_gather(g + LOOKAHEAD, bf)
        consume(g, b)

    for b in range(LOOKAHEAD):
        issue_gather(b, b)
    for g in range(NBUF):
        visit(g, g, drain_out=(g >= NBUF - LOOKAHEAD))

    def steady(i, carry):
        for b in range(NBUF):
            visit(i * NBUF + b, b, drain_out=True)
        return carry

    lax.fori_loop(1, G // NBUF - 1, steady, 0)

    for b in range(NBUF):
        g = G - NBUF + b
        if b < NBUF - LOOKAHEAD:
            visit(g, b, drain_out=True)
        else:
            consume(g, b)
    for b in range(NBUF):
        wait_out(b)


def kernel(x, table):
    xi = x.astype(jnp.int32).reshape(NW * IDX_ROWS, IDX_MINOR)
    out = _embed_sc(xi, table)
    return out.reshape(x.shape[0], x.shape[1], D_MODEL)


# speedup vs baseline: 1.1147x; 1.0014x over previous
import functools

import jax
import jax.numpy as jnp
from jax import lax
from jax.experimental import pallas as pl
from jax.experimental.pallas import tpu as pltpu
from jax.experimental.pallas import tpu_sc as plsc

D_MODEL = 64
SCALE = 8.0

NC, NS, LANES = 2, 16, 16
NW = NC * NS
XROWS, XCOLS = 4096, 200
RPW = XROWS // NW
SPLIT = 128
REST = XCOLS - SPLIT
NBUF = 4
LOOKAHEAD = 2

_mesh = plsc.VectorSubcoreMesh(core_axis_name="c", subcore_axis_name="s")


@functools.partial(
    pl.kernel,
    out_type=jax.ShapeDtypeStruct((XROWS, XCOLS, D_MODEL), jnp.float32),
    mesh=_mesh,
    scratch_types=[
        pltpu.VMEM((RPW, XCOLS), jnp.int32),
        [pltpu.VMEM((XCOLS, D_MODEL), jnp.float32)] * NBUF,
        [pltpu.SemaphoreType.DMA] * NBUF,
        [pltpu.SemaphoreType.DMA] * NBUF,
    ],
    compiler_params=pltpu.CompilerParams(use_tc_tiling_on_sc=False),
)
def _embed_sc(x_hbm, tab_hbm, out_hbm, idx_v, bufs, gsems, osems):
    wid = lax.axis_index("s") * NC + lax.axis_index("c")
    row0 = wid * RPW
    pltpu.sync_copy(x_hbm.at[pl.ds(row0, RPW)], idx_v)

    def issue_gather(r, b):
        pltpu.async_copy(
            tab_hbm.at[idx_v.at[r, pl.ds(0, SPLIT)]],
            bufs[b].at[pl.ds(0, SPLIT)],
            gsems[b],
        )
        pltpu.async_copy(
            tab_hbm.at[idx_v.at[r, pl.ds(SPLIT, REST)]],
            bufs[b].at[pl.ds(SPLIT, REST)],
            gsems[b],
        )

    def wait_gather(b):
        pltpu.make_async_copy(
            tab_hbm.at[idx_v.at[0, pl.ds(0, SPLIT)]],
            bufs[b].at[pl.ds(0, SPLIT)],
            gsems[b],
        ).wait()
        pltpu.make_async_copy(
            tab_hbm.at[idx_v.at[0, pl.ds(SPLIT, REST)]],
            bufs[b].at[pl.ds(SPLIT, REST)],
            gsems[b],
        ).wait()

    def wait_out(b):
        pltpu.make_async_copy(bufs[b], out_hbm.at[0], osems[b]).wait()

    def scale_buf(b):
        @plsc.parallel_loop(0, XCOLS, step=4)
        def _scale(r):
            for rr in range(4):
                for j in range(D_MODEL // LANES):
                    sl = pl.ds(j * LANES, LANES)
                    bufs[b][r + rr, sl] = bufs[b][r + rr, sl] * SCALE

    def consume(r, b):
        wait_gather(b)
        scale_buf(b)
        pltpu.async_copy(bufs[b], out_hbm.at[row0 + r], osems[b])

    def visit(r, b, drain_out):
        bf = (b + LOOKAHEAD) % NBUF
        if drain_out:
            wait_out(bf)
        issue_gather(r + LOOKAHEAD, bf)
        consume(r, b)

    for b in range(LOOKAHEAD):
        issue_gather(b, b)
    for r in range(NBUF):
        visit(r, r, drain_out=(r >= NBUF - LOOKAHEAD))

    def steady(i, carry):
        for b in range(NBUF):
            visit(i * NBUF + b, b, drain_out=True)
        return carry

    lax.fori_loop(1, RPW // NBUF - 1, steady, 0)

    for b in range(NBUF):
        r = RPW - NBUF + b
        if b < NBUF - LOOKAHEAD:
            visit(r, b, drain_out=True)
        else:
            consume(r, b)
    for b in range(NBUF):
        wait_out(b)


def kernel(x, table):
    return _embed_sc(x.astype(jnp.int32), table)
